# baseline (speedup 1.0000x reference)
import functools

import jax
import jax.numpy as jnp
from jax import lax
from jax.experimental import pallas as pl
from jax.experimental.pallas import tpu as pltpu
from jax.experimental.pallas import tpu_sc as plsc

_N = 4096
_NC = 2
_NS = 16
_NW = _NC * _NS
_RPW = _N // _NW
_CH = 8
_NCH = _RPW // _CH
_L = 16

_mesh = plsc.VectorSubcoreMesh(core_axis_name="c", subcore_axis_name="s")


@functools.partial(
    pl.kernel,
    out_type=jax.ShapeDtypeStruct((_N,), jnp.float32),
    mesh=_mesh,
    scratch_types=[
        pltpu.VMEM((_N,), jnp.float32),
        pltpu.VMEM((_CH, _N), jnp.float32),
        pltpu.VMEM((_CH, _N), jnp.float32),
        pltpu.VMEM((_RPW,), jnp.float32),
        pltpu.SemaphoreType.DMA,
        pltpu.SemaphoreType.DMA,
    ],
)
def _sis_sc(x_hbm, a_hbm, out_hbm, x_v, a0, a1, y_v, sem0, sem1):
    wid = lax.axis_index("s") * _NC + lax.axis_index("c")
    base = wid * _RPW
    pltpu.sync_copy(x_hbm, x_v)
    lane = lax.iota(jnp.int32, _L)
    zero = jnp.zeros((_L,), jnp.float32)

    def chunk_sums(buf):
        @pl.loop(0, _N // _L, init_carry=(zero,) * _CH, unroll=4)
        def accs(jb, accs):
            off = jb * _L
            xc = x_v[pl.ds(off, _L)]
            return tuple(
                accs[r] + buf[r, pl.ds(off, _L)] * xc for r in range(_CH)
            )

        sums = []
        for r in range(_CH):
            tot = accs[r]
            for m in (1, 2, 4, 8):
                tot = tot + tot.at[lane ^ m].get(
                    mode="promise_in_bounds", unique_indices=True)
            sums.append(tot)
        return sums

    pltpu.async_copy(a_hbm.at[pl.ds(base, _CH)], a0, sem0)

    @pl.loop(0, _NCH, step=2)
    def _pair(c):
        pltpu.make_async_copy(a_hbm.at[pl.ds(base, _CH)], a0, sem0).wait()
        pltpu.async_copy(a_hbm.at[pl.ds(base + (c + 1) * _CH, _CH)], a1, sem1)
        s0 = chunk_sums(a0)

        pltpu.make_async_copy(a_hbm.at[pl.ds(base, _CH)], a1, sem1).wait()

        @pl.when(c + 2 < _NCH)
        def _():
            pltpu.async_copy(
                a_hbm.at[pl.ds(base + (c + 2) * _CH, _CH)], a0, sem0)

        s1 = chunk_sums(a1)

        yv = zero
        for r in range(_CH):
            yv = jnp.where(lane == r, s0[r], yv)
            yv = jnp.where(lane == _CH + r, s1[r], yv)
        y_v[pl.ds(c * _CH, _L)] = yv

    for u in range(_RPW // _L):
        xr = x_v[pl.ds(base + u * _L, _L)]
        y_v[pl.ds(u * _L, _L)] = (1.0 - xr) * y_v[pl.ds(u * _L, _L)] - xr
    pltpu.sync_copy(y_v, out_hbm.at[pl.ds(base, _RPW)])


def kernel(t, x, A):
    return _sis_sc(x.reshape(_N), A).reshape(_N, 1)


# speedup vs baseline: 1.1637x; 1.1637x over previous
import functools

import jax
import jax.numpy as jnp
from jax import lax
from jax.experimental import pallas as pl
from jax.experimental.pallas import tpu as pltpu
from jax.experimental.pallas import tpu_sc as plsc

_N = 4096
_S = 2048
_NC = 2
_NS = 16
_NW = _NC * _NS
_RPW = _S // _NW
_CH = 8
_NCH = _RPW // _CH
_L = 16
_BM = 512

_mesh = plsc.VectorSubcoreMesh(core_axis_name="c", subcore_axis_name="s")


@functools.partial(
    pl.kernel,
    out_type=jax.ShapeDtypeStruct((_S,), jnp.float32),
    mesh=_mesh,
    scratch_types=[
        pltpu.VMEM((_N,), jnp.float32),
        pltpu.VMEM((_CH, _N), jnp.float32),
        pltpu.VMEM((_CH, _N), jnp.float32),
        pltpu.VMEM((_RPW,), jnp.float32),
        pltpu.SemaphoreType.DMA,
        pltpu.SemaphoreType.DMA,
    ],
)
def _sis_sc(x_hbm, a_hbm, out_hbm, x_v, a0, a1, y_v, sem0, sem1):
    wid = lax.axis_index("s") * _NC + lax.axis_index("c")
    base = wid * _RPW
    pltpu.sync_copy(x_hbm, x_v)
    lane = lax.iota(jnp.int32, _L)
    zero = jnp.zeros((_L,), jnp.float32)

    def chunk_sums(buf):
        @pl.loop(0, _N // _L, init_carry=(zero,) * _CH)
        def accs(jb, accs):
            off = jb * _L
            xc = x_v[pl.ds(off, _L)]
            return tuple(
                accs[r] + buf[r, pl.ds(off, _L)] * xc for r in range(_CH)
            )

        sums = []
        for r in range(_CH):
            tot = accs[r]
            for m in (1, 2, 4, 8):
                tot = tot + tot.at[lane ^ m].get(
                    mode="promise_in_bounds", unique_indices=True)
            sums.append(tot)
        return sums

    pltpu.async_copy(a_hbm.at[pl.ds(base, _CH)], a0, sem0)

    @pl.loop(0, _NCH, step=2)
    def _pair(c):
        pltpu.make_async_copy(a_hbm.at[pl.ds(base, _CH)], a0, sem0).wait()
        pltpu.async_copy(a_hbm.at[pl.ds(base + (c + 1) * _CH, _CH)], a1, sem1)
        s0 = chunk_sums(a0)

        pltpu.make_async_copy(a_hbm.at[pl.ds(base, _CH)], a1, sem1).wait()

        @pl.when(c + 2 < _NCH)
        def _():
            pltpu.async_copy(
                a_hbm.at[pl.ds(base + (c + 2) * _CH, _CH)], a0, sem0)

        s1 = chunk_sums(a1)

        yv = zero
        for r in range(_CH):
            yv = jnp.where(lane == r, s0[r], yv)
            yv = jnp.where(lane == _CH + r, s1[r], yv)
        y_v[pl.ds(c * _CH, _L)] = yv

    for u in range(_RPW // _L):
        xr = x_v[pl.ds(base + u * _L, _L)]
        y_v[pl.ds(u * _L, _L)] = (1.0 - xr) * y_v[pl.ds(u * _L, _L)] - xr
    pltpu.sync_copy(y_v, out_hbm.at[pl.ds(base, _RPW)])


def _sis_tc(a_ref, x_ref, xb_ref, o_ref):
    y = jnp.dot(a_ref[...], x_ref[...], preferred_element_type=jnp.float32)
    xb = xb_ref[...]
    o_ref[...] = (1.0 - xb) * y - xb


def kernel(t, x, A):
    f_sc = _sis_sc(x.reshape(_N), A)
    f_tc = pl.pallas_call(
        _sis_tc,
        grid=((_N - _S) // _BM,),
        in_specs=[
            pl.BlockSpec((_BM, _N), lambda i: (i + _S // _BM, 0)),
            pl.BlockSpec((_N, 1), lambda i: (0, 0)),
            pl.BlockSpec((_BM, 1), lambda i: (i + _S // _BM, 0)),
        ],
        out_specs=pl.BlockSpec((_BM, 1), lambda i: (i, 0)),
        out_shape=jax.ShapeDtypeStruct((_N - _S, 1), jnp.float32),
    )(A, x, x)
    return jnp.concatenate([f_sc.reshape(_S, 1), f_tc], axis=0)
